# Initial kernel scaffold; baseline (speedup 1.0000x reference)
import functools

import jax
import jax.numpy as jnp
from jax.experimental import pallas as pl
from jax.experimental.pallas import tpu as pltpu
from jax.experimental.pallas import tpu_sc as plsc

NU = 25000
NN = 50000
D = 64
HD = 32
B = 4096
NE = 800000
NEP = 819200
ROWS = NEP // 128
RPT = ROWS // 16
ACC = 51200
TRASH = NN
LAM = 0.001

_MESH = plsc.VectorSubcoreMesh(core_axis_name="c", subcore_axis_name="s")


def _deg(srcd2, zpat8, onepat):
    @functools.partial(
        pl.kernel,
        out_type=jax.ShapeDtypeStruct((2 * ACC, 8), jnp.float32),
        mesh=_MESH,
        scratch_types=[
            pltpu.VMEM((8, 128), jnp.int32),
            pltpu.VMEM((128, 8), jnp.float32),
            pltpu.VMEM((128, 8), jnp.float32),
            pltpu.VMEM_SHARED((ACC, 8), jnp.float32),
        ],
    )
    def k(src_hbm, zpat_hbm, one_hbm, out_hbm, sidx, zbuf, ones, acc):
        c = jax.lax.axis_index("c")
        s = jax.lax.axis_index("s")
        pltpu.sync_copy(zpat_hbm, zbuf)
        pltpu.sync_copy(one_hbm, ones)

        @pl.loop(0, 25)
        def _(i):
            pltpu.sync_copy(zbuf, acc.at[pl.ds(s * 3200 + i * 128, 128)])

        plsc.subcore_barrier()

        @pl.loop(0, 25)
        def _(b):
            r0 = c * 3200 + s * 200 + b * 8
            pltpu.sync_copy(src_hbm.at[pl.ds(r0, 8)], sidx)
            for j in range(8):
                pltpu.sync_copy(ones, acc.at[sidx.at[j]], add=True)

        plsc.subcore_barrier()

        @pl.loop(0, 25)
        def _(i):
            r = s * 3200 + i * 128
            pltpu.sync_copy(acc.at[pl.ds(r, 128)],
                            out_hbm.at[pl.ds(c * ACC + r, 128)])

    return k(srcd2, zpat8, onepat)


def _layer(hp, src3f, dstp2, zpat32):
    @functools.partial(
        pl.kernel,
        out_type=jax.ShapeDtypeStruct((2 * NN, HD), jnp.float32),
        mesh=_MESH,
        scratch_types=[
            pltpu.VMEM((8, 128), jnp.int32),
            pltpu.VMEM((8, 128), jnp.int32),
            pltpu.VMEM((128, HD), jnp.float32),
            pltpu.VMEM((128, HD), jnp.float32),
            pltpu.VMEM((128, HD), jnp.float32),
            pltpu.VMEM_SHARED((ACC, HD), jnp.float32),
            pltpu.SemaphoreType.DMA,
        ],
    )
    def k(hp_hbm, src_hbm, dst_hbm, zpat_hbm, out_hbm,
          sidx, didx, rows0, rows1, zbuf, acc, gsem):
        c = jax.lax.axis_index("c")
        s = jax.lax.axis_index("s")
        pltpu.sync_copy(zpat_hbm, zbuf)

        @pl.loop(0, 25)
        def _(i):
            pltpu.sync_copy(zbuf, acc.at[pl.ds(s * 3200 + i * 128, 128)])

        plsc.subcore_barrier()

        rowbufs = (rows0, rows1)

        @pl.loop(0, 50)
        def _(b):
            r0 = c * ROWS + s * RPT + b * 8
            pltpu.sync_copy(src_hbm.at[pl.ds(r0, 8)], sidx)
            pltpu.sync_copy(dst_hbm.at[pl.ds(s * RPT + b * 8, 8)], didx)
            cp = pltpu.async_copy(hp_hbm.at[sidx.at[0]], rows0, gsem)
            for j in range(8):
                cp.wait()
                if j < 7:
                    cp = pltpu.async_copy(
                        hp_hbm.at[sidx.at[j + 1]], rowbufs[(j + 1) % 2], gsem)
                pltpu.sync_copy(rowbufs[j % 2], acc.at[didx.at[j]], add=True)

        plsc.subcore_barrier()

        @pl.loop(0, 25)
        def _(i):
            pltpu.sync_copy(
                acc.at[pl.ds(s * 3125 + i * 125, 125)],
                out_hbm.at[pl.ds(c * NN + s * 3125 + i * 125, 125)])

    return k(hp, src3f, dstp2, zpat32)


def _batch_gather(sump, nrm8, gidx2, nidx2):
    @functools.partial(
        pl.kernel,
        out_type=(jax.ShapeDtypeStruct((6 * B, HD), jnp.float32),
                  jax.ShapeDtypeStruct((3 * B, 8), jnp.float32)),
        mesh=_MESH,
        scratch_types=[
            pltpu.VMEM((6, 128), jnp.int32),
            pltpu.VMEM((3, 128), jnp.int32),
            pltpu.VMEM((128, HD), jnp.float32),
            pltpu.VMEM((128, 8), jnp.float32),
            pltpu.SemaphoreType.DMA,
        ],
    )
    def k(sum_hbm, nrm_hbm, gi_hbm, ni_hbm, go_hbm, no_hbm,
          gi, ni, rows, nrows, sem):
        c = jax.lax.axis_index("c")
        s = jax.lax.axis_index("s")
        w = c * 16 + s
        pltpu.sync_copy(gi_hbm.at[pl.ds(w * 6, 6)], gi)
        pltpu.sync_copy(ni_hbm.at[pl.ds(w * 3, 3)], ni)
        for j in range(6):
            pltpu.async_copy(sum_hbm.at[gi.at[j]], rows, sem).wait()
            pltpu.sync_copy(rows, go_hbm.at[pl.ds(w * 768 + j * 128, 128)])
        for j in range(3):
            pltpu.async_copy(nrm_hbm.at[ni.at[j]], nrows, sem).wait()
            pltpu.sync_copy(nrows, no_hbm.at[pl.ds(w * 384 + j * 128, 128)])

    return k(sump, nrm8, gidx2, nidx2)


def _tc_prep(degp, ep3):
    def body(deg_ref, e_ref, sqdm_ref, h0_ref, nrm_ref):
        dg = jnp.sum(deg_ref[...], axis=(0, 2))
        sd = jnp.where(dg > 0, jax.lax.rsqrt(jnp.maximum(dg, 1.0)), 0.0)
        e = e_ref[...]
        sd3 = jnp.broadcast_to(sd[None, :, None], (2, 1000, HD))
        sqdm_ref[...] = sd3
        h0_ref[...] = e * sd3
        nrm_ref[...] = jnp.broadcast_to(
            jnp.sum(e * e, axis=(0, 2))[:, None], (1000, 8))

    return pl.pallas_call(
        body,
        grid=(50,),
        in_specs=[
            pl.BlockSpec((2, 1000, 8), lambda i: (0, i, 0)),
            pl.BlockSpec((2, 1000, HD), lambda i: (0, i, 0)),
        ],
        out_specs=[
            pl.BlockSpec((2, 1000, HD), lambda i: (0, i, 0)),
            pl.BlockSpec((2, 1000, HD), lambda i: (0, i, 0)),
            pl.BlockSpec((1000, 8), lambda i: (i, 0)),
        ],
        out_shape=[
            jax.ShapeDtypeStruct((2, NN, HD), jnp.float32),
            jax.ShapeDtypeStruct((2, NN, HD), jnp.float32),
            jax.ShapeDtypeStruct((NN, 8), jnp.float32),
        ],
    )(degp, ep3)


def _tc_scale(agg3, sqdm, sumin):
    def body(a_ref, q_ref, s_ref, h_ref, so_ref):
        a = a_ref[...]
        q = q_ref[...]
        ego = q * a
        so_ref[...] = s_ref[...] + ego
        h_ref[...] = q * ego

    bs = pl.BlockSpec((2, 1000, HD), lambda i: (0, i, 0))
    return pl.pallas_call(
        body,
        grid=(50,),
        in_specs=[bs, bs, bs],
        out_specs=[bs, bs],
        out_shape=[
            jax.ShapeDtypeStruct((2, NN, HD), jnp.float32),
            jax.ShapeDtypeStruct((2, NN, HD), jnp.float32),
        ],
    )(agg3, sqdm, sumin)


def _tc_final(grows, nvals):
    def body(g_ref, n_ref, o_ref):
        g = g_ref[...]
        ps = jnp.sum(g[0] * g[2] + g[1] * g[3], axis=1) * (1.0 / 16.0)
        ns = jnp.sum(g[0] * g[4] + g[1] * g[5], axis=1) * (1.0 / 16.0)
        x = ns - ps
        sp = jnp.maximum(x, 0.0) + jnp.log(1.0 + jnp.exp(-jnp.abs(x)))
        loss = jnp.sum(sp) * (1.0 / B)
        nv = n_ref[...]
        col0 = jax.lax.broadcasted_iota(jnp.int32, (3 * B, 8), 1) == 0
        reg = jnp.sum(jnp.where(col0, nv, 0.0)) * (1.0 / B)
        o_ref[0, 0] = loss + LAM * reg

    return pl.pallas_call(
        body,
        out_shape=jax.ShapeDtypeStruct((1, 1), jnp.float32),
    )(grows, nvals)


def kernel(users, pos, neg, edge_index, node_ids, emb_table):
    del node_ids
    users = users.astype(jnp.int32)
    pos = pos.astype(jnp.int32)
    neg = neg.astype(jnp.int32)
    src = edge_index[0].astype(jnp.int32)
    dst = edge_index[1].astype(jnp.int32)

    pad = NEP - NE
    srcg = jnp.concatenate([src, jnp.zeros((pad,), jnp.int32)])
    srcd2 = jnp.concatenate(
        [src, jnp.full((pad,), TRASH, jnp.int32)]).reshape(ROWS, 128)
    src3f = jnp.concatenate([srcg, srcg + NN]).reshape(2 * ROWS, 128)
    dstp2 = jnp.concatenate(
        [dst, jnp.full((pad,), TRASH, jnp.int32)]).reshape(ROWS, 128)

    ep = jnp.concatenate([emb_table[:, :HD], emb_table[:, HD:]], axis=0)
    ep3 = ep.reshape(2, NN, HD)

    zpat8 = jnp.zeros((128, 8), jnp.float32)
    zpat32 = jnp.zeros((128, HD), jnp.float32)
    onepat = jnp.where(
        jax.lax.broadcasted_iota(jnp.int32, (128, 8), 1) == 0, 1.0, 0.0)

    degp = _deg(srcd2, zpat8, onepat).reshape(2, ACC, 8)
    sqdm, h, nrm8 = _tc_prep(degp, ep3)

    summ = ep3
    for _ in range(3):
        agg = _layer(h.reshape(2 * NN, HD), src3f, dstp2, zpat32)
        h, summ = _tc_scale(agg.reshape(2, NN, HD), sqdm, summ)

    gidx2 = jnp.concatenate(
        [users, users + NN, pos + NU, pos + NU + NN,
         neg + NU, neg + NU + NN]).reshape(192, 128)
    nidx2 = jnp.concatenate([users, pos + NU, neg + NU]).reshape(96, 128)

    grows, nvals = _batch_gather(summ.reshape(2 * NN, HD), nrm8, gidx2, nidx2)
    out = _tc_final(grows.reshape(6, B, HD), nvals)
    return out[0, 0]


# speedup vs baseline: 6.7071x; 6.7071x over previous
import functools

import jax
import jax.numpy as jnp
from jax.experimental import pallas as pl
from jax.experimental.pallas import tpu as pltpu
from jax.experimental.pallas import tpu_sc as plsc

NU = 25000
NN = 50000
D = 64
HD = 32
B = 4096
NE = 800000
NEP = 819200
ROWS = NEP // 128
RPT = ROWS // 16
ACC = 51200
TRASH = NN
LAM = 0.001

_MESH = plsc.VectorSubcoreMesh(core_axis_name="c", subcore_axis_name="s")
_SC_PARAMS = pltpu.CompilerParams(use_tc_tiling_on_sc=False)


def _deg(srcd2, zpat8, onepat):
    @functools.partial(
        pl.kernel,
        out_type=jax.ShapeDtypeStruct((2 * ACC, 8), jnp.float32),
        mesh=_MESH,
        compiler_params=_SC_PARAMS,
        scratch_types=[
            pltpu.VMEM((8, 128), jnp.int32),
            pltpu.VMEM((128, 8), jnp.float32),
            pltpu.VMEM((128, 8), jnp.float32),
            pltpu.VMEM_SHARED((ACC, 8), jnp.float32),
        ],
    )
    def k(src_hbm, zpat_hbm, one_hbm, out_hbm, sidx, zbuf, ones, acc):
        c = jax.lax.axis_index("c")
        s = jax.lax.axis_index("s")
        pltpu.sync_copy(zpat_hbm, zbuf)
        pltpu.sync_copy(one_hbm, ones)

        @pl.loop(0, 25)
        def _(i):
            pltpu.sync_copy(zbuf, acc.at[pl.ds(s * 3200 + i * 128, 128)])

        plsc.subcore_barrier()

        @pl.loop(0, 25)
        def _(b):
            r0 = c * 3200 + s * 200 + b * 8
            pltpu.sync_copy(src_hbm.at[pl.ds(r0, 8)], sidx)
            for j in range(8):
                pltpu.sync_copy(ones, acc.at[sidx.at[j]], add=True)

        plsc.subcore_barrier()

        @pl.loop(0, 25)
        def _(i):
            r = s * 3200 + i * 128
            pltpu.sync_copy(acc.at[pl.ds(r, 128)],
                            out_hbm.at[pl.ds(c * ACC + r, 128)])

    return k(srcd2, zpat8, onepat)


def _layer(hp, src3f, dstp2, zpat32):
    @functools.partial(
        pl.kernel,
        out_type=jax.ShapeDtypeStruct((2 * ACC, HD), jnp.float32),
        mesh=_MESH,
        compiler_params=_SC_PARAMS,
        scratch_types=[
            pltpu.VMEM((8, 128), jnp.int32),
            pltpu.VMEM((8, 128), jnp.int32),
            pltpu.VMEM((128, HD), jnp.float32),
            pltpu.VMEM((128, HD), jnp.float32),
            pltpu.VMEM((128, HD), jnp.float32),
            pltpu.VMEM_SHARED((ACC, HD), jnp.float32),
            pltpu.SemaphoreType.DMA,
        ],
    )
    def k(hp_hbm, src_hbm, dst_hbm, zpat_hbm, out_hbm,
          sidx, didx, rows0, rows1, zbuf, acc, gsem):
        c = jax.lax.axis_index("c")
        s = jax.lax.axis_index("s")
        pltpu.sync_copy(zpat_hbm, zbuf)

        @pl.loop(0, 25)
        def _(i):
            pltpu.sync_copy(zbuf, acc.at[pl.ds(s * 3200 + i * 128, 128)])

        plsc.subcore_barrier()

        rowbufs = (rows0, rows1)

        @pl.loop(0, 50)
        def _(b):
            r0 = c * ROWS + s * RPT + b * 8
            pltpu.sync_copy(src_hbm.at[pl.ds(r0, 8)], sidx)
            pltpu.sync_copy(dst_hbm.at[pl.ds(s * RPT + b * 8, 8)], didx)
            cp = pltpu.async_copy(hp_hbm.at[sidx.at[0]], rows0, gsem)
            for j in range(8):
                cp.wait()
                if j < 7:
                    cp = pltpu.async_copy(
                        hp_hbm.at[sidx.at[j + 1]], rowbufs[(j + 1) % 2], gsem)
                pltpu.sync_copy(rowbufs[j % 2], acc.at[didx.at[j]], add=True)

        plsc.subcore_barrier()

        @pl.loop(0, 25)
        def _(i):
            r = s * 3200 + i * 128
            pltpu.sync_copy(acc.at[pl.ds(r, 128)],
                            out_hbm.at[pl.ds(c * ACC + r, 128)])

    return k(hp, src3f, dstp2, zpat32)


def _batch_gather(sump, nrm8, gidx2, nidx2):
    @functools.partial(
        pl.kernel,
        out_type=(jax.ShapeDtypeStruct((6 * B, HD), jnp.float32),
                  jax.ShapeDtypeStruct((3 * B, 8), jnp.float32)),
        mesh=_MESH,
        compiler_params=_SC_PARAMS,
        scratch_types=[
            pltpu.VMEM((1, 6, 128), jnp.int32),
            pltpu.VMEM((1, 3, 128), jnp.int32),
            pltpu.VMEM((128, HD), jnp.float32),
            pltpu.VMEM((128, 8), jnp.float32),
            pltpu.SemaphoreType.DMA,
        ],
    )
    def k(sum_hbm, nrm_hbm, gi_hbm, ni_hbm, go_hbm, no_hbm,
          gi, ni, rows, nrows, sem):
        c = jax.lax.axis_index("c")
        s = jax.lax.axis_index("s")
        w = c * 16 + s
        pltpu.sync_copy(gi_hbm.at[pl.ds(w, 1)], gi)
        pltpu.sync_copy(ni_hbm.at[pl.ds(w, 1)], ni)
        for j in range(6):
            pltpu.async_copy(sum_hbm.at[gi.at[0, j]], rows, sem).wait()
            pltpu.sync_copy(rows, go_hbm.at[pl.ds(w * 768 + j * 128, 128)])
        for j in range(3):
            pltpu.async_copy(nrm_hbm.at[ni.at[0, j]], nrows, sem).wait()
            pltpu.sync_copy(nrows, no_hbm.at[pl.ds(w * 384 + j * 128, 128)])

    return k(sump, nrm8, gidx2, nidx2)


def _tc_prep(degp, ep3):
    def body(deg_ref, e_ref, sqdm_ref, h0_ref, nrm_ref):
        dg = jnp.sum(deg_ref[...], axis=(0, 2))
        sd = jnp.where(dg > 0, jax.lax.rsqrt(jnp.maximum(dg, 1.0)), 0.0)
        e = e_ref[...]
        sd3 = jnp.broadcast_to(sd[None, :, None], (2, 1000, HD))
        sqdm_ref[...] = sd3
        h0_ref[...] = e * sd3
        nrm_ref[...] = jnp.broadcast_to(
            jnp.sum(e * e, axis=(0, 2))[:, None], (1000, 8))

    return pl.pallas_call(
        body,
        grid=(50,),
        in_specs=[
            pl.BlockSpec((2, 1000, 8), lambda i: (0, i, 0)),
            pl.BlockSpec((2, 1000, HD), lambda i: (0, i, 0)),
        ],
        out_specs=[
            pl.BlockSpec((2, 1000, HD), lambda i: (0, i, 0)),
            pl.BlockSpec((2, 1000, HD), lambda i: (0, i, 0)),
            pl.BlockSpec((1000, 8), lambda i: (i, 0)),
        ],
        out_shape=[
            jax.ShapeDtypeStruct((2, NN, HD), jnp.float32),
            jax.ShapeDtypeStruct((2, NN, HD), jnp.float32),
            jax.ShapeDtypeStruct((NN, 8), jnp.float32),
        ],
    )(degp, ep3)


def _tc_scale(agg3, sqdm, sumin):
    def body(a_ref, q_ref, s_ref, h_ref, so_ref):
        a = a_ref[...]
        q = q_ref[...]
        ego = q * a
        so_ref[...] = s_ref[...] + ego
        h_ref[...] = q * ego

    bs = pl.BlockSpec((2, 1000, HD), lambda i: (0, i, 0))
    return pl.pallas_call(
        body,
        grid=(50,),
        in_specs=[bs, bs, bs],
        out_specs=[bs, bs],
        out_shape=[
            jax.ShapeDtypeStruct((2, NN, HD), jnp.float32),
            jax.ShapeDtypeStruct((2, NN, HD), jnp.float32),
        ],
    )(agg3, sqdm, sumin)


def _tc_final(grows, nvals):
    def body(g_ref, n_ref, o_ref):
        g = g_ref[...]
        ps = jnp.sum(g[0] * g[2] + g[1] * g[3], axis=1) * (1.0 / 16.0)
        ns = jnp.sum(g[0] * g[4] + g[1] * g[5], axis=1) * (1.0 / 16.0)
        x = ns - ps
        sp = jnp.maximum(x, 0.0) + jnp.log(1.0 + jnp.exp(-jnp.abs(x)))
        loss = jnp.sum(sp) * (1.0 / B)
        nv = n_ref[...]
        col0 = jax.lax.broadcasted_iota(jnp.int32, (3 * B, 8), 1) == 0
        reg = jnp.sum(jnp.where(col0, nv, 0.0)) * (1.0 / B)
        o_ref[...] = jnp.broadcast_to(loss + LAM * reg, (1, 1))

    return pl.pallas_call(
        body,
        out_shape=jax.ShapeDtypeStruct((1, 1), jnp.float32),
    )(grows, nvals)


def kernel(users, pos, neg, edge_index, node_ids, emb_table):
    del node_ids
    users = users.astype(jnp.int32)
    pos = pos.astype(jnp.int32)
    neg = neg.astype(jnp.int32)
    src = edge_index[0].astype(jnp.int32)
    dst = edge_index[1].astype(jnp.int32)

    pad = NEP - NE
    srcg = jnp.concatenate([src, jnp.zeros((pad,), jnp.int32)])
    srcd2 = jnp.concatenate(
        [src, jnp.full((pad,), TRASH, jnp.int32)]).reshape(ROWS, 128)
    src3f = jnp.concatenate([srcg, srcg + NN]).reshape(2 * ROWS, 128)
    dstp2 = jnp.concatenate(
        [dst, jnp.full((pad,), TRASH, jnp.int32)]).reshape(ROWS, 128)

    ep = jnp.concatenate([emb_table[:, :HD], emb_table[:, HD:]], axis=0)
    ep3 = ep.reshape(2, NN, HD)

    zpat8 = jnp.zeros((128, 8), jnp.float32)
    zpat32 = jnp.zeros((128, HD), jnp.float32)
    onepat = jnp.where(
        jax.lax.broadcasted_iota(jnp.int32, (128, 8), 1) == 0, 1.0, 0.0)

    degp = _deg(srcd2, zpat8, onepat).reshape(2, ACC, 8)
    sqdm, h, nrm8 = _tc_prep(degp, ep3)

    summ = ep3
    for _ in range(3):
        agg = _layer(h.reshape(2 * NN, HD), src3f, dstp2, zpat32)
        h, summ = _tc_scale(agg.reshape(2, ACC, HD), sqdm, summ)

    gidx2 = jnp.concatenate(
        [users, users + NN, pos + NU, pos + NU + NN,
         neg + NU, neg + NU + NN]).reshape(32, 6, 128)
    nidx2 = jnp.concatenate([users, pos + NU, neg + NU]).reshape(32, 3, 128)

    grows, nvals = _batch_gather(summ.reshape(2 * NN, HD), nrm8, gidx2, nidx2)
    out = _tc_final(grows.reshape(6, B, HD), nvals)
    return out[0, 0]


# speedup vs baseline: 8.3537x; 1.2455x over previous
import functools

import jax
import jax.numpy as jnp
from jax.experimental import pallas as pl
from jax.experimental.pallas import tpu as pltpu
from jax.experimental.pallas import tpu_sc as plsc

NU = 25000
NN = 50000
D = 64
HD = 32
B = 4096
NE = 800000
NEP = 819200
ROWS = NEP // 128
RPT = ROWS // 16
ACC = 51200
TRASH = NN
LAM = 0.001

_MESH = plsc.VectorSubcoreMesh(core_axis_name="c", subcore_axis_name="s")
_SC_PARAMS = pltpu.CompilerParams(use_tc_tiling_on_sc=False)


def _deg(srcd2, zpat8, onepat):
    @functools.partial(
        pl.kernel,
        out_type=jax.ShapeDtypeStruct((2 * ACC, 8), jnp.float32),
        mesh=_MESH,
        compiler_params=_SC_PARAMS,
        scratch_types=[
            pltpu.VMEM((8, 128), jnp.int32),
            pltpu.VMEM((128, 8), jnp.float32),
            pltpu.VMEM((128, 8), jnp.float32),
            pltpu.VMEM_SHARED((ACC, 8), jnp.float32),
        ],
    )
    def k(src_hbm, zpat_hbm, one_hbm, out_hbm, sidx, zbuf, ones, acc):
        c = jax.lax.axis_index("c")
        s = jax.lax.axis_index("s")
        pltpu.sync_copy(zpat_hbm, zbuf)
        pltpu.sync_copy(one_hbm, ones)

        @pl.loop(0, 25)
        def _(i):
            pltpu.sync_copy(zbuf, acc.at[pl.ds(s * 3200 + i * 128, 128)])

        plsc.subcore_barrier()

        @pl.loop(0, 25)
        def _(b):
            r0 = c * 3200 + s * 200 + b * 8
            pltpu.sync_copy(src_hbm.at[pl.ds(r0, 8)], sidx)
            for j in range(8):
                pltpu.sync_copy(ones, acc.at[sidx.at[j]], add=True)

        plsc.subcore_barrier()

        @pl.loop(0, 25)
        def _(i):
            r = s * 3200 + i * 128
            pltpu.sync_copy(acc.at[pl.ds(r, 128)],
                            out_hbm.at[pl.ds(c * ACC + r, 128)])

    return k(srcd2, zpat8, onepat)


def _layer(hp, src3f, dstp2, zpat32):
    @functools.partial(
        pl.kernel,
        out_type=jax.ShapeDtypeStruct((2 * ACC, HD), jnp.float32),
        mesh=_MESH,
        compiler_params=_SC_PARAMS,
        scratch_types=[
            pltpu.VMEM((8, 128), jnp.int32),
            pltpu.VMEM((8, 128), jnp.int32),
            pltpu.VMEM((8, 128), jnp.int32),
            pltpu.VMEM((8, 128), jnp.int32),
            pltpu.VMEM((128, HD), jnp.float32),
            pltpu.VMEM((128, HD), jnp.float32),
            pltpu.VMEM((128, HD), jnp.float32),
            pltpu.VMEM((128, HD), jnp.float32),
            pltpu.VMEM_SHARED((ACC, HD), jnp.float32),
        ] + [pltpu.SemaphoreType.DMA] * 7,
    )
    def k(hp_hbm, src_hbm, dst_hbm, zpat_hbm, out_hbm,
          sidxA, didxA, sidxB, didxB, b0, b1, b2, b3, acc, *sems):
        gsemA, gsemB, ssemA, ssemB, isemA, isemB, zsem = sems
        c = jax.lax.axis_index("c")
        s = jax.lax.axis_index("s")
        idxg = ((sidxA, didxA, isemA), (sidxB, didxB, isemB))
        bufg = ((b0, b1, gsemA, ssemA), (b2, b3, gsemB, ssemB))

        pltpu.sync_copy(zpat_hbm, b0)
        for i in range(25):
            pltpu.async_copy(b0, acc.at[pl.ds(s * 3200 + i * 128, 128)],
                             zsem)
        for i in range(25):
            pltpu.make_async_copy(
                b0, acc.at[pl.ds(s * 3200 + i * 128, 128)], zsem).wait()
        plsc.subcore_barrier()

        def fire_idx(i8, x):
            sidx, didx, isem = idxg[x]
            r0 = s * RPT + i8 * 8
            pltpu.async_copy(src_hbm.at[pl.ds(c * ROWS + r0, 8)], sidx, isem)
            pltpu.async_copy(dst_hbm.at[pl.ds(r0, 8)], didx, isem)

        def drain_idx(x):
            sidx, didx, isem = idxg[x]
            pltpu.make_async_copy(src_hbm.at[pl.ds(0, 8)], sidx, isem).wait()
            pltpu.make_async_copy(dst_hbm.at[pl.ds(0, 8)], didx, isem).wait()

        def fire_gathers(x, j0, g):
            sidx = idxg[x][0]
            ba, bb, gsem, _ = bufg[g]
            pltpu.async_copy(hp_hbm.at[sidx.at[j0]], ba, gsem)
            pltpu.async_copy(hp_hbm.at[sidx.at[j0 + 1]], bb, gsem)

        def drain_gathers(g):
            ba, bb, gsem, _ = bufg[g]
            pltpu.make_async_copy(hp_hbm.at[sidxA.at[0]], ba, gsem).wait()
            pltpu.make_async_copy(hp_hbm.at[sidxA.at[0]], bb, gsem).wait()

        def fire_scatters(x, j0, g):
            didx = idxg[x][1]
            ba, bb, _, ssem = bufg[g]
            pltpu.async_copy(ba, acc.at[didx.at[j0]], ssem, add=True)
            pltpu.async_copy(bb, acc.at[didx.at[j0 + 1]], ssem, add=True)

        def drain_scatters(g):
            ba, bb, _, ssem = bufg[g]
            pltpu.make_async_copy(ba, acc.at[didxA.at[0]], ssem).wait()
            pltpu.make_async_copy(bb, acc.at[didxA.at[1]], ssem).wait()

        fire_idx(0, 0)
        drain_idx(0)
        fire_gathers(0, 0, 0)

        @pl.loop(0, 25)
        def _(i2):
            for x in (0, 1):
                i = i2 * 2 + x

                @pl.when(i < 49)
                def _():
                    fire_idx(i + 1, 1 - x)

                for kk in range(4):
                    g = kk % 2
                    drain_gathers(g)
                    fire_scatters(x, 2 * kk, g)
                    if kk == 0:
                        @pl.when(i > 0)
                        def _():
                            drain_scatters(1)
                    else:
                        drain_scatters(1 - g)
                    if kk < 3:
                        fire_gathers(x, 2 * kk + 2, 1 - g)
                    else:
                        @pl.when(i < 49)
                        def _():
                            drain_idx(1 - x)
                            fire_gathers(1 - x, 0, 0)

        drain_scatters(1)

        plsc.subcore_barrier()

        wcps = [pltpu.async_copy(
            acc.at[pl.ds(s * 3200 + i * 128, 128)],
            out_hbm.at[pl.ds(c * ACC + s * 3200 + i * 128, 128)], zsem)
            for i in range(25)]
        for cp in wcps:
            cp.wait()

    return k(hp, src3f, dstp2, zpat32)


def _batch_gather(sump, nrm8, gidx2, nidx2):
    @functools.partial(
        pl.kernel,
        out_type=(jax.ShapeDtypeStruct((6 * B, HD), jnp.float32),
                  jax.ShapeDtypeStruct((3 * B, 8), jnp.float32)),
        mesh=_MESH,
        compiler_params=_SC_PARAMS,
        scratch_types=[
            pltpu.VMEM((1, 6, 128), jnp.int32),
            pltpu.VMEM((1, 3, 128), jnp.int32),
            pltpu.VMEM((128, HD), jnp.float32),
            pltpu.VMEM((128, 8), jnp.float32),
            pltpu.SemaphoreType.DMA,
        ],
    )
    def k(sum_hbm, nrm_hbm, gi_hbm, ni_hbm, go_hbm, no_hbm,
          gi, ni, rows, nrows, sem):
        c = jax.lax.axis_index("c")
        s = jax.lax.axis_index("s")
        w = c * 16 + s
        pltpu.sync_copy(gi_hbm.at[pl.ds(w, 1)], gi)
        pltpu.sync_copy(ni_hbm.at[pl.ds(w, 1)], ni)
        for j in range(6):
            pltpu.async_copy(sum_hbm.at[gi.at[0, j]], rows, sem).wait()
            pltpu.sync_copy(rows, go_hbm.at[pl.ds(w * 768 + j * 128, 128)])
        for j in range(3):
            pltpu.async_copy(nrm_hbm.at[ni.at[0, j]], nrows, sem).wait()
            pltpu.sync_copy(nrows, no_hbm.at[pl.ds(w * 384 + j * 128, 128)])

    return k(sump, nrm8, gidx2, nidx2)


def _tc_prep(degp, ep3):
    def body(deg_ref, e_ref, sqdm_ref, h0_ref, nrm_ref):
        dg = jnp.sum(deg_ref[...], axis=(0, 2))
        sd = jnp.where(dg > 0, jax.lax.rsqrt(jnp.maximum(dg, 1.0)), 0.0)
        e = e_ref[...]
        sd3 = jnp.broadcast_to(sd[None, :, None], (2, 1000, HD))
        sqdm_ref[...] = sd3
        h0_ref[...] = e * sd3
        nrm_ref[...] = jnp.broadcast_to(
            jnp.sum(e * e, axis=(0, 2))[:, None], (1000, 8))

    return pl.pallas_call(
        body,
        grid=(50,),
        in_specs=[
            pl.BlockSpec((2, 1000, 8), lambda i: (0, i, 0)),
            pl.BlockSpec((2, 1000, HD), lambda i: (0, i, 0)),
        ],
        out_specs=[
            pl.BlockSpec((2, 1000, HD), lambda i: (0, i, 0)),
            pl.BlockSpec((2, 1000, HD), lambda i: (0, i, 0)),
            pl.BlockSpec((1000, 8), lambda i: (i, 0)),
        ],
        out_shape=[
            jax.ShapeDtypeStruct((2, NN, HD), jnp.float32),
            jax.ShapeDtypeStruct((2, NN, HD), jnp.float32),
            jax.ShapeDtypeStruct((NN, 8), jnp.float32),
        ],
    )(degp, ep3)


def _tc_scale(agg3, sqdm, sumin):
    def body(a_ref, q_ref, s_ref, h_ref, so_ref):
        a = a_ref[...]
        q = q_ref[...]
        ego = q * a
        so_ref[...] = s_ref[...] + ego
        h_ref[...] = q * ego

    bs = pl.BlockSpec((2, 1000, HD), lambda i: (0, i, 0))
    return pl.pallas_call(
        body,
        grid=(50,),
        in_specs=[bs, bs, bs],
        out_specs=[bs, bs],
        out_shape=[
            jax.ShapeDtypeStruct((2, NN, HD), jnp.float32),
            jax.ShapeDtypeStruct((2, NN, HD), jnp.float32),
        ],
    )(agg3, sqdm, sumin)


def _tc_final(grows, nvals):
    def body(g_ref, n_ref, o_ref):
        g = g_ref[...]
        ps = jnp.sum(g[0] * g[2] + g[1] * g[3], axis=1) * (1.0 / 16.0)
        ns = jnp.sum(g[0] * g[4] + g[1] * g[5], axis=1) * (1.0 / 16.0)
        x = ns - ps
        sp = jnp.maximum(x, 0.0) + jnp.log(1.0 + jnp.exp(-jnp.abs(x)))
        loss = jnp.sum(sp) * (1.0 / B)
        nv = n_ref[...]
        col0 = jax.lax.broadcasted_iota(jnp.int32, (3 * B, 8), 1) == 0
        reg = jnp.sum(jnp.where(col0, nv, 0.0)) * (1.0 / B)
        o_ref[...] = jnp.broadcast_to(loss + LAM * reg, (1, 1))

    return pl.pallas_call(
        body,
        out_shape=jax.ShapeDtypeStruct((1, 1), jnp.float32),
    )(grows, nvals)


def kernel(users, pos, neg, edge_index, node_ids, emb_table):
    del node_ids
    users = users.astype(jnp.int32)
    pos = pos.astype(jnp.int32)
    neg = neg.astype(jnp.int32)
    src = edge_index[0].astype(jnp.int32)
    dst = edge_index[1].astype(jnp.int32)

    pad = NEP - NE
    srcg = jnp.concatenate([src, jnp.zeros((pad,), jnp.int32)])
    srcd2 = jnp.concatenate(
        [src, jnp.full((pad,), TRASH, jnp.int32)]).reshape(ROWS, 128)
    src3f = jnp.concatenate([srcg, srcg + NN]).reshape(2 * ROWS, 128)
    dstp2 = jnp.concatenate(
        [dst, jnp.full((pad,), TRASH, jnp.int32)]).reshape(ROWS, 128)

    ep = jnp.concatenate([emb_table[:, :HD], emb_table[:, HD:]], axis=0)
    ep3 = ep.reshape(2, NN, HD)

    zpat8 = jnp.zeros((128, 8), jnp.float32)
    zpat32 = jnp.zeros((128, HD), jnp.float32)
    onepat = jnp.where(
        jax.lax.broadcasted_iota(jnp.int32, (128, 8), 1) == 0, 1.0, 0.0)

    degp = _deg(srcd2, zpat8, onepat).reshape(2, ACC, 8)
    sqdm, h, nrm8 = _tc_prep(degp, ep3)

    summ = ep3
    for _ in range(3):
        agg = _layer(h.reshape(2 * NN, HD), src3f, dstp2, zpat32)
        h, summ = _tc_scale(agg.reshape(2, ACC, HD), sqdm, summ)

    gidx2 = jnp.concatenate(
        [users, users + NN, pos + NU, pos + NU + NN,
         neg + NU, neg + NU + NN]).reshape(32, 6, 128)
    nidx2 = jnp.concatenate([users, pos + NU, neg + NU]).reshape(32, 3, 128)

    grows, nvals = _batch_gather(summ.reshape(2 * NN, HD), nrm8, gidx2, nidx2)
    out = _tc_final(grows.reshape(6, B, HD), nvals)
    return out[0, 0]


# speedup vs baseline: 11.0489x; 1.3226x over previous
import functools

import jax
import jax.numpy as jnp
from jax.experimental import pallas as pl
from jax.experimental.pallas import tpu as pltpu
from jax.experimental.pallas import tpu_sc as plsc

NU = 25000
NN = 50000
D = 64
HD = 32
B = 4096
NE = 800000
EW = 96
NEP = 811008
ROWS = NEP // EW
RPT = ROWS // 16
NB = RPT // 8
ACC = 51200
TRASH = NN
LAM = 0.001
RING = 8
STAG = 5

_MESH = plsc.VectorSubcoreMesh(core_axis_name="c", subcore_axis_name="s")
_SC_PARAMS = pltpu.CompilerParams(use_tc_tiling_on_sc=False)


def _deg(srcd2, zpat8, onepat):
    @functools.partial(
        pl.kernel,
        out_type=jax.ShapeDtypeStruct((2 * ACC, 8), jnp.float32),
        mesh=_MESH,
        compiler_params=_SC_PARAMS,
        scratch_types=[
            pltpu.VMEM((8, EW), jnp.int32),
            pltpu.VMEM((8, EW), jnp.int32),
            pltpu.VMEM((128, 8), jnp.float32),
            pltpu.VMEM((EW, 8), jnp.float32),
            pltpu.VMEM_SHARED((ACC, 8), jnp.float32),
        ] + [pltpu.SemaphoreType.DMA] * 3,
    )
    def k(src_hbm, zpat_hbm, one_hbm, out_hbm, sidxA, sidxB, zbuf, ones,
          acc, isem, ssem, zsem):
        c = jax.lax.axis_index("c")
        s = jax.lax.axis_index("s")
        pltpu.sync_copy(zpat_hbm, zbuf)
        pltpu.sync_copy(one_hbm, ones)
        for i in range(25):
            pltpu.async_copy(zbuf, acc.at[pl.ds(s * 3200 + i * 128, 128)],
                             zsem)
        for i in range(25):
            pltpu.make_async_copy(
                zbuf, acc.at[pl.ds(s * 3200 + i * 128, 128)], zsem).wait()
        plsc.subcore_barrier()

        sidxg = (sidxA, sidxB)

        def fire_idx(blk, x):
            r0 = c * (ROWS // 2) + s * (RPT // 2) + blk * 8
            pltpu.async_copy(src_hbm.at[pl.ds(r0, 8)], sidxg[x], isem)

        def drain_idx(x):
            pltpu.make_async_copy(src_hbm.at[pl.ds(0, 8)], sidxg[x],
                                  isem).wait()

        def body(blk, x, prefetch):
            drain_idx(x)
            if prefetch:
                fire_idx(blk + 1, 1 - x)
            for j in range(8):
                pltpu.async_copy(ones, acc.at[sidxg[x].at[j]], ssem,
                                 add=True)
            for j in range(8):
                pltpu.make_async_copy(ones, acc.at[sidxg[x].at[j]],
                                      ssem).wait()

        fire_idx(0, 0)

        @pl.loop(0, 16)
        def _(t):
            body(2 * t, 0, True)
            body(2 * t + 1, 1, True)

        body(32, 0, False)

        plsc.subcore_barrier()

        @pl.loop(0, 25)
        def _(i):
            r = s * 3200 + i * 128
            pltpu.sync_copy(acc.at[pl.ds(r, 128)],
                            out_hbm.at[pl.ds(c * ACC + r, 128)])

    return k(srcd2, zpat8, onepat)


def _layer(hp, src3f, dstp2, zpat32):
    buf_types = [pltpu.VMEM((EW, HD), jnp.float32) for _ in range(RING)]

    @functools.partial(
        pl.kernel,
        out_type=jax.ShapeDtypeStruct((2 * ACC, HD), jnp.float32),
        mesh=_MESH,
        compiler_params=_SC_PARAMS,
        scratch_types=[
            pltpu.VMEM((8, EW), jnp.int32),
            pltpu.VMEM((8, EW), jnp.int32),
            pltpu.VMEM((8, EW), jnp.int32),
            pltpu.VMEM((8, EW), jnp.int32),
            pltpu.VMEM_SHARED((ACC, HD), jnp.float32),
        ] + buf_types + [pltpu.SemaphoreType.DMA] * 5,
    )
    def k(hp_hbm, src_hbm, dst_hbm, zpat_hbm, out_hbm,
          sidxA, didxA, sidxB, didxB, acc, *rest):
        bufs = rest[:RING]
        gsem, ssem, isemA, isemB, zsem = rest[RING:]
        c = jax.lax.axis_index("c")
        s = jax.lax.axis_index("s")
        idxg = ((sidxA, didxA, isemA), (sidxB, didxB, isemB))

        pltpu.sync_copy(zpat_hbm, bufs[0])
        zslices = [(i * EW, EW) for i in range(33)] + [(33 * EW, 32)]
        for off, ln in zslices:
            pltpu.async_copy(bufs[0].at[pl.ds(0, ln)],
                             acc.at[pl.ds(s * 3200 + off, ln)], zsem)
        for off, ln in zslices:
            pltpu.make_async_copy(
                bufs[0].at[pl.ds(0, ln)],
                acc.at[pl.ds(s * 3200 + off, ln)], zsem).wait()
        plsc.subcore_barrier()

        def fire_idx(blk, x):
            sidx, didx, isem = idxg[x]
            r0 = s * RPT + blk * 8
            pltpu.async_copy(src_hbm.at[pl.ds(c * ROWS + r0, 8)], sidx, isem)
            pltpu.async_copy(dst_hbm.at[pl.ds(r0, 8)], didx, isem)

        def drain_idx(x):
            sidx, didx, isem = idxg[x]
            pltpu.make_async_copy(src_hbm.at[pl.ds(0, 8)], sidx, isem).wait()
            pltpu.make_async_copy(dst_hbm.at[pl.ds(0, 8)], didx, isem).wait()

        def fire_gather(x, j):
            pltpu.async_copy(hp_hbm.at[idxg[x][0].at[j]], bufs[j], gsem)

        def drain_gather(q):
            pltpu.make_async_copy(hp_hbm.at[sidxA.at[0]], bufs[q],
                                  gsem).wait()

        def fire_scatter(x, jrow, q):
            pltpu.async_copy(bufs[q], acc.at[idxg[x][1].at[jrow]], ssem,
                             add=True)

        def drain_scatter(q):
            pltpu.make_async_copy(bufs[q], acc.at[didxA.at[0]], ssem).wait()

        def step(j, x, blk, first_blk, last_blk):
            if not first_blk:
                drain_scatter(j)
            if not last_blk and ((first_blk and j == 5) or
                                 (not first_blk and j == 7)):
                fire_idx(blk + 1, 1 - x)
            fire_gather(x, j)
            q = (j + RING - STAG) % RING
            if first_blk:
                if j >= STAG:
                    drain_gather(q)
                    fire_scatter(x, j - STAG, q)
            else:
                drain_gather(q)
                if j < STAG:
                    fire_scatter(1 - x, j + 8 - STAG, q)
                else:
                    fire_scatter(x, j - STAG, q)

        fire_idx(0, 0)
        drain_idx(0)
        for j in range(8):
            step(j, 0, 0, True, False)

        @pl.loop(0, (NB - 2) // 2)
        def _(t):
            blk = 1 + 2 * t
            drain_idx(1)
            for j in range(8):
                step(j, 1, blk, False, False)
            drain_idx(0)
            for j in range(8):
                step(j, 0, blk + 1, False, False)

        drain_idx(1)
        for j in range(8):
            step(j, 1, NB - 1, False, True)

        for t in range(STAG):
            q = (8 - STAG + t) % RING
            drain_gather(q)
            fire_scatter(1, 8 - STAG + t, q)
        for q in range(RING):
            drain_scatter(q)

        plsc.subcore_barrier()

        wcps = [pltpu.async_copy(
            acc.at[pl.ds(s * 3200 + i * 128, 128)],
            out_hbm.at[pl.ds(c * ACC + s * 3200 + i * 128, 128)], zsem)
            for i in range(25)]
        for cp in wcps:
            cp.wait()

    return k(hp, src3f, dstp2, zpat32)


def _batch_gather(sump, nrm8, gidx2, nidx2):
    @functools.partial(
        pl.kernel,
        out_type=(jax.ShapeDtypeStruct((6 * B, HD), jnp.float32),
                  jax.ShapeDtypeStruct((3 * B, 8), jnp.float32)),
        mesh=_MESH,
        compiler_params=_SC_PARAMS,
        scratch_types=[
            pltpu.VMEM((1, 6, 128), jnp.int32),
            pltpu.VMEM((1, 3, 128), jnp.int32),
            pltpu.VMEM((128, HD), jnp.float32),
            pltpu.VMEM((128, 8), jnp.float32),
            pltpu.SemaphoreType.DMA,
        ],
    )
    def k(sum_hbm, nrm_hbm, gi_hbm, ni_hbm, go_hbm, no_hbm,
          gi, ni, rows, nrows, sem):
        c = jax.lax.axis_index("c")
        s = jax.lax.axis_index("s")
        w = c * 16 + s
        pltpu.sync_copy(gi_hbm.at[pl.ds(w, 1)], gi)
        pltpu.sync_copy(ni_hbm.at[pl.ds(w, 1)], ni)
        for j in range(6):
            pltpu.async_copy(sum_hbm.at[gi.at[0, j]], rows, sem).wait()
            pltpu.sync_copy(rows, go_hbm.at[pl.ds(w * 768 + j * 128, 128)])
        for j in range(3):
            pltpu.async_copy(nrm_hbm.at[ni.at[0, j]], nrows, sem).wait()
            pltpu.sync_copy(nrows, no_hbm.at[pl.ds(w * 384 + j * 128, 128)])

    return k(sump, nrm8, gidx2, nidx2)


def _tc_prep(degp, ep3):
    def body(deg_ref, e_ref, sqdm_ref, h0_ref, nrm_ref):
        dg = jnp.sum(deg_ref[...], axis=(0, 2))
        sd = jnp.where(dg > 0, jax.lax.rsqrt(jnp.maximum(dg, 1.0)), 0.0)
        e = e_ref[...]
        sd3 = jnp.broadcast_to(sd[None, :, None], (2, 1000, HD))
        sqdm_ref[...] = sd3
        h0_ref[...] = e * sd3
        nrm_ref[...] = jnp.broadcast_to(
            jnp.sum(e * e, axis=(0, 2))[:, None], (1000, 8))

    return pl.pallas_call(
        body,
        grid=(50,),
        in_specs=[
            pl.BlockSpec((2, 1000, 8), lambda i: (0, i, 0)),
            pl.BlockSpec((2, 1000, HD), lambda i: (0, i, 0)),
        ],
        out_specs=[
            pl.BlockSpec((2, 1000, HD), lambda i: (0, i, 0)),
            pl.BlockSpec((2, 1000, HD), lambda i: (0, i, 0)),
            pl.BlockSpec((1000, 8), lambda i: (i, 0)),
        ],
        out_shape=[
            jax.ShapeDtypeStruct((2, NN, HD), jnp.float32),
            jax.ShapeDtypeStruct((2, NN, HD), jnp.float32),
            jax.ShapeDtypeStruct((NN, 8), jnp.float32),
        ],
    )(degp, ep3)


def _tc_scale(agg3, sqdm, sumin):
    def body(a_ref, q_ref, s_ref, h_ref, so_ref):
        a = a_ref[...]
        q = q_ref[...]
        ego = q * a
        so_ref[...] = s_ref[...] + ego
        h_ref[...] = q * ego

    bs = pl.BlockSpec((2, 1000, HD), lambda i: (0, i, 0))
    return pl.pallas_call(
        body,
        grid=(50,),
        in_specs=[bs, bs, bs],
        out_specs=[bs, bs],
        out_shape=[
            jax.ShapeDtypeStruct((2, NN, HD), jnp.float32),
            jax.ShapeDtypeStruct((2, NN, HD), jnp.float32),
        ],
    )(agg3, sqdm, sumin)


def _tc_final(grows, nvals):
    def body(g_ref, n_ref, o_ref):
        g = g_ref[...]
        ps = jnp.sum(g[0] * g[2] + g[1] * g[3], axis=1) * (1.0 / 16.0)
        ns = jnp.sum(g[0] * g[4] + g[1] * g[5], axis=1) * (1.0 / 16.0)
        x = ns - ps
        sp = jnp.maximum(x, 0.0) + jnp.log(1.0 + jnp.exp(-jnp.abs(x)))
        loss = jnp.sum(sp) * (1.0 / B)
        nv = n_ref[...]
        col0 = jax.lax.broadcasted_iota(jnp.int32, (3 * B, 8), 1) == 0
        reg = jnp.sum(jnp.where(col0, nv, 0.0)) * (1.0 / B)
        o_ref[...] = jnp.broadcast_to(loss + LAM * reg, (1, 1))

    return pl.pallas_call(
        body,
        out_shape=jax.ShapeDtypeStruct((1, 1), jnp.float32),
    )(grows, nvals)


def kernel(users, pos, neg, edge_index, node_ids, emb_table):
    del node_ids
    users = users.astype(jnp.int32)
    pos = pos.astype(jnp.int32)
    neg = neg.astype(jnp.int32)
    src = edge_index[0].astype(jnp.int32)
    dst = edge_index[1].astype(jnp.int32)

    pad = NEP - NE
    srcg = jnp.concatenate([src, jnp.zeros((pad,), jnp.int32)])
    srcd2 = jnp.concatenate(
        [src, jnp.full((pad,), TRASH, jnp.int32)]).reshape(ROWS, EW)
    src3f = jnp.concatenate([srcg, srcg + NN]).reshape(2 * ROWS, EW)
    dstp2 = jnp.concatenate(
        [dst, jnp.full((pad,), TRASH, jnp.int32)]).reshape(ROWS, EW)

    ep = jnp.concatenate([emb_table[:, :HD], emb_table[:, HD:]], axis=0)
    ep3 = ep.reshape(2, NN, HD)

    zpat8 = jnp.zeros((128, 8), jnp.float32)
    zpat32 = jnp.zeros((EW, HD), jnp.float32)
    onepat = jnp.where(
        jax.lax.broadcasted_iota(jnp.int32, (EW, 8), 1) == 0, 1.0, 0.0)

    degp = _deg(srcd2, zpat8, onepat).reshape(2, ACC, 8)
    sqdm, h, nrm8 = _tc_prep(degp, ep3)

    summ = ep3
    for _ in range(3):
        agg = _layer(h.reshape(2 * NN, HD), src3f, dstp2, zpat32)
        h, summ = _tc_scale(agg.reshape(2, ACC, HD), sqdm, summ)

    gidx2 = jnp.concatenate(
        [users, users + NN, pos + NU, pos + NU + NN,
         neg + NU, neg + NU + NN]).reshape(32, 6, 128)
    nidx2 = jnp.concatenate([users, pos + NU, neg + NU]).reshape(32, 3, 128)

    grows, nvals = _batch_gather(summ.reshape(2 * NN, HD), nrm8, gidx2, nidx2)
    out = _tc_final(grows.reshape(6, B, HD), nvals)
    return out[0, 0]


# speedup vs baseline: 12.2455x; 1.1083x over previous
import functools

import jax
import jax.numpy as jnp
from jax.experimental import pallas as pl
from jax.experimental.pallas import tpu as pltpu
from jax.experimental.pallas import tpu_sc as plsc

NU = 25000
NN = 50000
D = 64
HD = 32
B = 4096
NE = 800000
EW = 96
NEP = 811008
ROWS = NEP // EW
RPT = ROWS // 16
NB = RPT // 8
ACC = 51200
TRASH = NN
LAM = 0.001
RING = 8
STAG = 5

_MESH = plsc.VectorSubcoreMesh(core_axis_name="c", subcore_axis_name="s")
_SC_PARAMS = pltpu.CompilerParams(use_tc_tiling_on_sc=False)


def _deg(srcd2, zpat8, onepat):
    @functools.partial(
        pl.kernel,
        out_type=jax.ShapeDtypeStruct((2 * ACC, 8), jnp.float32),
        mesh=_MESH,
        compiler_params=_SC_PARAMS,
        scratch_types=[
            pltpu.VMEM((8, EW), jnp.int32),
            pltpu.VMEM((8, EW), jnp.int32),
            pltpu.VMEM((128, 8), jnp.float32),
            pltpu.VMEM((EW, 8), jnp.float32),
            pltpu.VMEM_SHARED((ACC, 8), jnp.float32),
        ] + [pltpu.SemaphoreType.DMA] * 3,
    )
    def k(src_hbm, zpat_hbm, one_hbm, out_hbm, sidxA, sidxB, zbuf, ones,
          acc, isem, ssem, zsem):
        c = jax.lax.axis_index("c")
        s = jax.lax.axis_index("s")
        pltpu.sync_copy(zpat_hbm, zbuf)
        pltpu.sync_copy(one_hbm, ones)
        for i in range(25):
            pltpu.async_copy(zbuf, acc.at[pl.ds(s * 3200 + i * 128, 128)],
                             zsem)
        for i in range(25):
            pltpu.make_async_copy(
                zbuf, acc.at[pl.ds(s * 3200 + i * 128, 128)], zsem).wait()
        plsc.subcore_barrier()

        sidxg = (sidxA, sidxB)

        def fire_idx(blk, x):
            r0 = c * (ROWS // 2) + s * (RPT // 2) + blk * 8
            pltpu.async_copy(src_hbm.at[pl.ds(r0, 8)], sidxg[x], isem)

        def drain_idx(x):
            pltpu.make_async_copy(src_hbm.at[pl.ds(0, 8)], sidxg[x],
                                  isem).wait()

        def body(blk, x, prefetch):
            drain_idx(x)
            if prefetch:
                fire_idx(blk + 1, 1 - x)
            for j in range(8):
                pltpu.async_copy(ones, acc.at[sidxg[x].at[j]], ssem,
                                 add=True)
            for j in range(8):
                pltpu.make_async_copy(ones, acc.at[sidxg[x].at[j]],
                                      ssem).wait()

        fire_idx(0, 0)

        @pl.loop(0, 16)
        def _(t):
            body(2 * t, 0, True)
            body(2 * t + 1, 1, True)

        body(32, 0, False)

        plsc.subcore_barrier()

        @pl.loop(0, 25)
        def _(i):
            r = s * 3200 + i * 128
            pltpu.sync_copy(acc.at[pl.ds(r, 128)],
                            out_hbm.at[pl.ds(c * ACC + r, 128)])

    return k(srcd2, zpat8, onepat)


def _layer(hp, src3f, dstp2, zpat32, scale):
    buf_types = [pltpu.VMEM((EW, HD), jnp.float32) for _ in range(RING)]

    @functools.partial(
        pl.kernel,
        out_type=jax.ShapeDtypeStruct((2 * ACC, HD), jnp.float32),
        mesh=_MESH,
        compiler_params=_SC_PARAMS,
        scratch_types=[
            pltpu.VMEM((8, EW), jnp.int32),
            pltpu.VMEM((8, EW), jnp.int32),
            pltpu.VMEM((8, EW), jnp.int32),
            pltpu.VMEM((8, EW), jnp.int32),
            pltpu.VMEM_SHARED((ACC, HD), jnp.float32),
        ] + buf_types + [pltpu.SemaphoreType.DMA] * 5,
    )
    def k(hp_hbm, src_hbm, dst_hbm, zpat_hbm, q_hbm, out_hbm,
          sidxA, didxA, sidxB, didxB, acc, *rest):
        bufs = rest[:RING]
        gsem, ssem, isemA, isemB, zsem = rest[RING:]
        c = jax.lax.axis_index("c")
        s = jax.lax.axis_index("s")
        idxg = ((sidxA, didxA, isemA), (sidxB, didxB, isemB))

        pltpu.sync_copy(zpat_hbm, bufs[0])
        zslices = [(i * EW, EW) for i in range(33)] + [(33 * EW, 32)]
        for off, ln in zslices:
            pltpu.async_copy(bufs[0].at[pl.ds(0, ln)],
                             acc.at[pl.ds(s * 3200 + off, ln)], zsem)
        for off, ln in zslices:
            pltpu.make_async_copy(
                bufs[0].at[pl.ds(0, ln)],
                acc.at[pl.ds(s * 3200 + off, ln)], zsem).wait()
        plsc.subcore_barrier()

        def fire_idx(blk, x):
            sidx, didx, isem = idxg[x]
            r0 = s * RPT + blk * 8
            pltpu.async_copy(src_hbm.at[pl.ds(c * ROWS + r0, 8)], sidx, isem)
            pltpu.async_copy(dst_hbm.at[pl.ds(r0, 8)], didx, isem)

        def drain_idx(x):
            sidx, didx, isem = idxg[x]
            pltpu.make_async_copy(src_hbm.at[pl.ds(0, 8)], sidx, isem).wait()
            pltpu.make_async_copy(dst_hbm.at[pl.ds(0, 8)], didx, isem).wait()

        def fire_gather(x, j):
            pltpu.async_copy(hp_hbm.at[idxg[x][0].at[j]], bufs[j], gsem)

        def drain_gather(q):
            pltpu.make_async_copy(hp_hbm.at[sidxA.at[0]], bufs[q],
                                  gsem).wait()

        def fire_scatter(x, jrow, q):
            pltpu.async_copy(bufs[q], acc.at[idxg[x][1].at[jrow]], ssem,
                             add=True)

        def drain_scatter(q):
            pltpu.make_async_copy(bufs[q], acc.at[didxA.at[0]], ssem).wait()

        def step(j, x, blk, first_blk, last_blk):
            if not first_blk:
                drain_scatter(j)
            if not last_blk and ((first_blk and j == 5) or
                                 (not first_blk and j == 7)):
                fire_idx(blk + 1, 1 - x)
            fire_gather(x, j)
            q = (j + RING - STAG) % RING
            if first_blk:
                if j >= STAG:
                    drain_gather(q)
                    fire_scatter(x, j - STAG, q)
            else:
                drain_gather(q)
                if j < STAG:
                    fire_scatter(1 - x, j + 8 - STAG, q)
                else:
                    fire_scatter(x, j - STAG, q)

        fire_idx(0, 0)
        drain_idx(0)
        for j in range(8):
            step(j, 0, 0, True, False)

        @pl.loop(0, (NB - 2) // 2)
        def _(t):
            blk = 1 + 2 * t
            drain_idx(1)
            for j in range(8):
                step(j, 1, blk, False, False)
            drain_idx(0)
            for j in range(8):
                step(j, 0, blk + 1, False, False)

        drain_idx(1)
        for j in range(8):
            step(j, 1, NB - 1, False, True)

        for t in range(STAG):
            q = (8 - STAG + t) % RING
            drain_gather(q)
            fire_scatter(1, 8 - STAG + t, q)
        for q in range(RING):
            drain_scatter(q)

        plsc.subcore_barrier()

        qb, ab, eb = bufs[0], bufs[1], bufs[2]

        def scale_chunk(r0, ln):
            pltpu.sync_copy(q_hbm.at[pl.ds(r0, ln)], qb.at[pl.ds(0, ln)])
            pltpu.sync_copy(acc.at[pl.ds(r0, ln)], ab.at[pl.ds(0, ln)])

            @pl.loop(0, ln)
            def _(r):
                for hh in (0, 16):
                    slc = (pl.ds(r, 1), pl.ds(hh, 16))
                    eb.at[*slc][...] = ab.at[*slc][...] * qb.at[*slc][...]

            pltpu.sync_copy(eb.at[pl.ds(0, ln)],
                            out_hbm.at[pl.ds(c * ACC + r0, ln)])

        @pl.loop(0, 33)
        def _(ci):
            scale_chunk(s * 3200 + ci * EW, EW)

        scale_chunk(s * 3200 + 33 * EW, 32)

    return k(hp, src3f, dstp2, zpat32, scale)


def _batch_gather(ep2, h1, h2, e3, aux, gidx2, nidx2):
    @functools.partial(
        pl.kernel,
        out_type=(jax.ShapeDtypeStruct((4 * 6 * B, HD), jnp.float32),
                  jax.ShapeDtypeStruct((3 * B, 8), jnp.float32)),
        mesh=_MESH,
        compiler_params=_SC_PARAMS,
        scratch_types=[
            pltpu.VMEM((1, 6, 128), jnp.int32),
            pltpu.VMEM((1, 3, 128), jnp.int32),
            pltpu.VMEM((128, HD), jnp.float32),
            pltpu.VMEM((128, HD), jnp.float32),
            pltpu.VMEM((128, 8), jnp.float32),
            pltpu.SemaphoreType.DMA,
            pltpu.SemaphoreType.DMA,
        ],
    )
    def k(e_hbm, h1_hbm, h2_hbm, e3_hbm, aux_hbm, gi_hbm, ni_hbm,
          go_hbm, no_hbm, gi, ni, rows0, rows1, nrows, gsem, osem):
        c = jax.lax.axis_index("c")
        s = jax.lax.axis_index("s")
        w = c * 16 + s
        pltpu.sync_copy(gi_hbm.at[pl.ds(w, 1)], gi)
        pltpu.sync_copy(ni_hbm.at[pl.ds(w, 1)], ni)
        tables = (e_hbm, h1_hbm, h2_hbm, e3_hbm)
        rb = (rows0, rows1)
        work = [(t, j) for t in range(4) for j in range(6)]
        pltpu.async_copy(tables[0].at[gi.at[0, 0]], rows0, gsem)
        for i, (t, j) in enumerate(work):
            pltpu.make_async_copy(tables[t].at[gi.at[0, j]],
                                  rb[i % 2], gsem).wait()
            if i + 1 < len(work):
                t2, j2 = work[i + 1]
                pltpu.async_copy(tables[t2].at[gi.at[0, j2]],
                                 rb[(i + 1) % 2], gsem)
            pltpu.sync_copy(
                rb[i % 2],
                go_hbm.at[pl.ds(t * 6 * B + w * 768 + j * 128, 128)])
        for j in range(3):
            pltpu.async_copy(aux_hbm.at[ni.at[0, j]], nrows, gsem).wait()
            pltpu.sync_copy(nrows, no_hbm.at[pl.ds(w * 384 + j * 128, 128)])

    return k(ep2, h1, h2, e3, aux, gidx2, nidx2)


def _tc_prep(degp, ep3):
    def body(deg_ref, e_ref, h0_ref, qm_ref, q2m_ref, aux_ref):
        dg = jnp.sum(deg_ref[...], axis=(0, 2))
        sd = jnp.where(dg > 0, jax.lax.rsqrt(jnp.maximum(dg, 1.0)), 0.0)
        e = e_ref[...]
        sd3 = jnp.broadcast_to(sd[None, :, None], (2, 1024, HD))
        h0_ref[...] = e * sd3
        qm_ref[...] = jnp.broadcast_to(sd[:, None], (1024, HD))
        q2m_ref[...] = jnp.broadcast_to((sd * sd)[:, None], (1024, HD))
        nrm = jnp.sum(e * e, axis=(0, 2))
        col = jax.lax.broadcasted_iota(jnp.int32, (1024, 8), 1)
        aux_ref[...] = jnp.where(
            col == 0, nrm[:, None],
            jnp.where(col == 1, sd[:, None], 0.0))

    return pl.pallas_call(
        body,
        grid=(50,),
        in_specs=[
            pl.BlockSpec((2, 1024, 8), lambda i: (0, i, 0)),
            pl.BlockSpec((2, 1024, HD), lambda i: (0, i, 0)),
        ],
        out_specs=[
            pl.BlockSpec((2, 1024, HD), lambda i: (0, i, 0)),
            pl.BlockSpec((1024, HD), lambda i: (i, 0)),
            pl.BlockSpec((1024, HD), lambda i: (i, 0)),
            pl.BlockSpec((1024, 8), lambda i: (i, 0)),
        ],
        out_shape=[
            jax.ShapeDtypeStruct((2, ACC, HD), jnp.float32),
            jax.ShapeDtypeStruct((ACC, HD), jnp.float32),
            jax.ShapeDtypeStruct((ACC, HD), jnp.float32),
            jax.ShapeDtypeStruct((ACC, 8), jnp.float32),
        ],
    )(degp, ep3)


def _tc_final(grows, auxg):
    CB = 512

    def body(g_ref, a_ref, o_ref):
        g = g_ref[...]
        av = a_ref[...]
        col = jax.lax.broadcasted_iota(jnp.int32, (3, CB, 8), 2)
        reg = jnp.sum(jnp.where(col == 0, av, 0.0))
        sd = jnp.sum(jnp.where(col == 1, av, 0.0), axis=2)
        inv = jnp.where(sd > 0, 1.0 / sd, 0.0)

        def S(k):
            iv = inv[k // 2][:, None]
            return g[0, k] + (g[1, k] + g[2, k]) * iv + g[3, k]

        ps = jnp.sum(S(0) * S(2) + S(1) * S(3), axis=1) * (1.0 / 16.0)
        ns = jnp.sum(S(0) * S(4) + S(1) * S(5), axis=1) * (1.0 / 16.0)
        x = ns - ps
        sp = jnp.maximum(x, 0.0) + jnp.log(1.0 + jnp.exp(-jnp.abs(x)))
        part = (jnp.sum(sp) + LAM * reg) * (1.0 / B)

        @pl.when(pl.program_id(0) == 0)
        def _():
            o_ref[...] = jnp.zeros((1, 1), jnp.float32)

        o_ref[...] += jnp.broadcast_to(part, (1, 1))

    return pl.pallas_call(
        body,
        grid=(B // CB,),
        in_specs=[
            pl.BlockSpec((4, 6, CB, HD), lambda i: (0, 0, i, 0)),
            pl.BlockSpec((3, CB, 8), lambda i: (0, i, 0)),
        ],
        out_specs=pl.BlockSpec((1, 1), lambda i: (0, 0)),
        out_shape=jax.ShapeDtypeStruct((1, 1), jnp.float32),
    )(grows, auxg)


def kernel(users, pos, neg, edge_index, node_ids, emb_table):
    del node_ids
    users = users.astype(jnp.int32)
    pos = pos.astype(jnp.int32)
    neg = neg.astype(jnp.int32)
    src = edge_index[0].astype(jnp.int32)
    dst = edge_index[1].astype(jnp.int32)

    pad = NEP - NE
    srcg = jnp.concatenate([src, jnp.zeros((pad,), jnp.int32)])
    srcd2 = jnp.concatenate(
        [src, jnp.full((pad,), TRASH, jnp.int32)]).reshape(ROWS, EW)
    src3f = jnp.concatenate([srcg, srcg + ACC]).reshape(2 * ROWS, EW)
    dstp2 = jnp.concatenate(
        [dst, jnp.full((pad,), TRASH, jnp.int32)]).reshape(ROWS, EW)

    zrows = jnp.zeros((ACC - NN, HD), jnp.float32)
    ep2 = jnp.concatenate(
        [emb_table[:, :HD], zrows, emb_table[:, HD:], zrows], axis=0)
    ep3 = ep2.reshape(2, ACC, HD)

    zpat8 = jnp.zeros((128, 8), jnp.float32)
    zpat32 = jnp.zeros((EW, HD), jnp.float32)
    onepat = jnp.where(
        jax.lax.broadcasted_iota(jnp.int32, (EW, 8), 1) == 0, 1.0, 0.0)

    degp = _deg(srcd2, zpat8, onepat).reshape(2, ACC, 8)
    h0, qm, q2m, aux = _tc_prep(degp, ep3)

    h1 = _layer(h0.reshape(2 * ACC, HD), src3f, dstp2, zpat32, q2m)
    h2 = _layer(h1, src3f, dstp2, zpat32, q2m)
    e3 = _layer(h2, src3f, dstp2, zpat32, qm)

    gidx2 = jnp.concatenate(
        [users, users + ACC, pos + NU, pos + NU + ACC,
         neg + NU, neg + NU + ACC]).reshape(32, 6, 128)
    nidx2 = jnp.concatenate([users, pos + NU, neg + NU]).reshape(32, 3, 128)

    grows, auxg = _batch_gather(ep2, h1, h2, e3, aux, gidx2, nidx2)
    out = _tc_final(grows.reshape(4, 6, B, HD), auxg.reshape(3, B, 8))
    return out[0, 0]
